# baseline (speedup 1.0000x reference)
import functools

import jax
import jax.numpy as jnp
from jax import lax
from jax.experimental import pallas as pl
from jax.experimental.pallas import tpu as pltpu
from jax.experimental.pallas import tpu_sc as plsc

H = 1024
FFN = 2816
E = 16
TOPK = 2
FBLK = 1408
NF = FFN // FBLK
T = 64


def _route_kernel(x_ref, wr_ref, o_ref):
    o_ref[...] = jax.lax.dot_general(
        x_ref[...], wr_ref[...], (((1,), (1,)), ((), ())),
        preferred_element_type=jnp.float32)


def _make_gates_sc():
    info = plsc.get_sparse_core_info()
    nc, ns = info.num_cores, info.num_subcores
    nw = nc * ns
    tpw = T // nw
    mesh = plsc.VectorSubcoreMesh(core_axis_name="c", subcore_axis_name="s")

    @functools.partial(
        pl.kernel, mesh=mesh,
        out_type=jax.ShapeDtypeStruct((T, E), jnp.float32),
        scratch_types=[
            pltpu.VMEM((tpw, E), jnp.float32),
            pltpu.VMEM((tpw, E), jnp.float32),
        ],
    )
    def gates_sc(route_hbm, out_hbm, route_v, gates_v):
        wid = lax.axis_index("s") * nc + lax.axis_index("c")
        base = wid * tpw
        pltpu.sync_copy(route_hbm.at[pl.ds(base, tpw)], route_v)
        io = lax.iota(jnp.int32, 16)
        dnums = lax.GatherDimensionNumbers(
            offset_dims=(), collapsed_slice_dims=(0,), start_index_map=(0,))

        def _shuf(v, idx):
            return lax.gather(v, idx[:, None], dnums, (1,),
                              mode=lax.GatherScatterMode.PROMISE_IN_BOUNDS)

        def _allreduce(v, op):
            for k in (1, 2, 4, 8):
                v = op(v, _shuf(v, jnp.bitwise_xor(io, k)))
            return v

        for j in range(tpw):
            r = route_v[j]
            m1 = _allreduce(r, jnp.maximum)
            i1 = _allreduce(jnp.where(r == m1, io, E), jnp.minimum)
            r2 = jnp.where(io == i1, -jnp.inf, r)
            m2 = _allreduce(r2, jnp.maximum)
            i2 = _allreduce(jnp.where(r2 == m2, io, E), jnp.minimum)
            g1 = 1.0 / (1.0 + jnp.exp(m2 - m1))
            gates_v[j] = jnp.where(io == i1, g1, 0.0) + jnp.where(io == i2, 1.0 - g1, 0.0)
        pltpu.sync_copy(gates_v, out_hbm.at[pl.ds(base, tpw)])

    return gates_sc


def _moe_kernel(x_ref, gin_ref, w1a_ref, w1b_ref, w2_ref, out_ref, gates_ref):
    e = pl.program_id(0)
    f = pl.program_id(1)

    @pl.when(jnp.logical_and(e == 0, f == 0))
    def _transpose_gates():
        gates_ref[...] = gin_ref[...].T

    x = x_ref[...]
    w1a = w1a_ref[0]
    w1b = w1b_ref[0]
    x0 = jax.lax.dot_general(x, w1a, (((1,), (1,)), ((), ())),
                             preferred_element_type=jnp.float32)
    x1 = jax.lax.dot_general(x, w1b, (((1,), (1,)), ((), ())),
                             preferred_element_type=jnp.float32)
    act = x0 * (x1 * jax.nn.sigmoid(x1))
    y = jax.lax.dot_general(act, w2_ref[0], (((1,), (1,)), ((), ())),
                            preferred_element_type=jnp.float32)
    g = gates_ref[e, :][:, None]
    contrib = g * y

    @pl.when(jnp.logical_and(e == 0, f == 0))
    def _init():
        out_ref[...] = contrib

    @pl.when(jnp.logical_not(jnp.logical_and(e == 0, f == 0)))
    def _acc():
        out_ref[...] += contrib


@functools.partial(jax.jit, static_argnames=())
def kernel(hidden_states, Wr, W1, W2):
    s, b, n = hidden_states.shape
    x = hidden_states.reshape(-1, n)
    route = pl.pallas_call(
        _route_kernel,
        out_shape=jax.ShapeDtypeStruct((T, E), jnp.float32),
    )(x, Wr)
    gates = _make_gates_sc()(route)
    out = pl.pallas_call(
        _moe_kernel,
        grid=(E, NF),
        in_specs=[
            pl.BlockSpec((T, H), lambda e, f: (0, 0)),
            pl.BlockSpec((T, E), lambda e, f: (0, 0)),
            pl.BlockSpec((1, FBLK, H), lambda e, f: (e, f, 0)),
            pl.BlockSpec((1, FBLK, H), lambda e, f: (e, NF + f, 0)),
            pl.BlockSpec((1, H, FBLK), lambda e, f: (e, 0, f)),
        ],
        out_specs=pl.BlockSpec((T, H), lambda e, f: (0, 0)),
        out_shape=jax.ShapeDtypeStruct((T, H), jnp.float32),
        scratch_shapes=[pltpu.VMEM((E, T), jnp.float32)],
        compiler_params=pltpu.CompilerParams(
            dimension_semantics=("arbitrary", "arbitrary"),
        ),
    )(x, gates, W1, W1, W2)
    return out.reshape(s, b, n)


# speedup vs baseline: 1.1195x; 1.1195x over previous
import functools

import jax
import jax.numpy as jnp
from jax.experimental import pallas as pl
from jax.experimental.pallas import tpu as pltpu

H = 1024
FFN = 2816
E = 16
TOPK = 2
FBLK = 1408
NF = FFN // FBLK


def _moe_kernel(x_ref, wr_ref, w1a_ref, w1b_ref, w2_ref, out_ref, gates_ref):
    e = pl.program_id(0)
    f = pl.program_id(1)

    @pl.when(jnp.logical_and(e == 0, f == 0))
    def _compute_gates():
        x = x_ref[...]
        route = jax.lax.dot_general(
            x, wr_ref[...], (((1,), (1,)), ((), ())),
            preferred_element_type=jnp.float32)
        T = route.shape[0]
        lane = jax.lax.broadcasted_iota(jnp.int32, (T, E), 1)
        m1 = jnp.max(route, axis=1, keepdims=True)
        i1 = jnp.argmax(route, axis=1)[:, None]
        masked = jnp.where(lane == i1, -jnp.inf, route)
        m2 = jnp.max(masked, axis=1, keepdims=True)
        i2 = jnp.argmax(masked, axis=1)[:, None]
        g1 = jax.nn.sigmoid(m1 - m2)
        g2 = 1.0 - g1
        gates = jnp.where(lane == i1, g1, 0.0) + jnp.where(lane == i2, g2, 0.0)
        gates_ref[...] = gates.T

    x = x_ref[...]
    w1a = w1a_ref[0]
    w1b = w1b_ref[0]
    x0 = jax.lax.dot_general(x, w1a, (((1,), (1,)), ((), ())),
                             preferred_element_type=jnp.float32)
    x1 = jax.lax.dot_general(x, w1b, (((1,), (1,)), ((), ())),
                             preferred_element_type=jnp.float32)
    act = x0 * (x1 * jax.nn.sigmoid(x1))
    y = jax.lax.dot_general(act, w2_ref[0], (((1,), (1,)), ((), ())),
                            preferred_element_type=jnp.float32)
    g = gates_ref[e, :][:, None]
    contrib = g * y

    @pl.when(jnp.logical_and(e == 0, f == 0))
    def _init():
        out_ref[...] = contrib

    @pl.when(jnp.logical_not(jnp.logical_and(e == 0, f == 0)))
    def _acc():
        out_ref[...] += contrib


@functools.partial(jax.jit, static_argnames=())
def kernel(hidden_states, Wr, W1, W2):
    s, b, n = hidden_states.shape
    x = hidden_states.reshape(-1, n)
    T = x.shape[0]
    out = pl.pallas_call(
        _moe_kernel,
        grid=(E, NF),
        in_specs=[
            pl.BlockSpec((T, H), lambda e, f: (0, 0)),
            pl.BlockSpec((E, H), lambda e, f: (0, 0)),
            pl.BlockSpec((1, FBLK, H), lambda e, f: (e, f, 0)),
            pl.BlockSpec((1, FBLK, H), lambda e, f: (e, NF + f, 0)),
            pl.BlockSpec((1, H, FBLK), lambda e, f: (e, 0, f)),
        ],
        out_specs=pl.BlockSpec((T, H), lambda e, f: (0, 0)),
        out_shape=jax.ShapeDtypeStruct((T, H), jnp.float32),
        scratch_shapes=[pltpu.VMEM((E, T), jnp.float32)],
        compiler_params=pltpu.CompilerParams(
            dimension_semantics=("arbitrary", "arbitrary"),
        ),
    )(x, Wr, W1, W1, W2)
    return out.reshape(s, b, n)


# speedup vs baseline: 1.1242x; 1.0042x over previous
import functools

import jax
import jax.numpy as jnp
from jax.experimental import pallas as pl
from jax.experimental.pallas import tpu as pltpu

H = 1024
FFN = 2816
E = 16
TOPK = 2
FBLK = 1408
NF = FFN // FBLK


def _moe_kernel(x_ref, wr_ref, w1_ref, w2_ref, out_ref, gates_ref):
    e = pl.program_id(0)
    f = pl.program_id(1)

    @pl.when(jnp.logical_and(e == 0, f == 0))
    def _compute_gates():
        x = x_ref[...]
        route = jax.lax.dot_general(
            x, wr_ref[...], (((1,), (1,)), ((), ())),
            preferred_element_type=jnp.float32)
        T = route.shape[0]
        lane = jax.lax.broadcasted_iota(jnp.int32, (T, E), 1)
        m1 = jnp.max(route, axis=1, keepdims=True)
        i1 = jnp.argmax(route, axis=1)[:, None]
        masked = jnp.where(lane == i1, -jnp.inf, route)
        m2 = jnp.max(masked, axis=1, keepdims=True)
        i2 = jnp.argmax(masked, axis=1)[:, None]
        g1 = jax.nn.sigmoid(m1 - m2)
        g2 = 1.0 - g1
        gates = jnp.where(lane == i1, g1, 0.0) + jnp.where(lane == i2, g2, 0.0)
        gates_ref[...] = gates.T

    x = x_ref[...]
    w1a = w1_ref[0, 0]
    w1b = w1_ref[0, 1]
    x0 = jax.lax.dot_general(x, w1a, (((1,), (1,)), ((), ())),
                             preferred_element_type=jnp.float32)
    x1 = jax.lax.dot_general(x, w1b, (((1,), (1,)), ((), ())),
                             preferred_element_type=jnp.float32)
    act = x0 * (x1 * jax.nn.sigmoid(x1))
    y = jax.lax.dot_general(act, w2_ref[0], (((1,), (1,)), ((), ())),
                            preferred_element_type=jnp.float32)
    g = gates_ref[e, :][:, None]
    contrib = g * y

    @pl.when(jnp.logical_and(e == 0, f == 0))
    def _init():
        out_ref[...] = contrib

    @pl.when(jnp.logical_not(jnp.logical_and(e == 0, f == 0)))
    def _acc():
        out_ref[...] += contrib


@functools.partial(jax.jit, static_argnames=())
def kernel(hidden_states, Wr, W1, W2):
    s, b, n = hidden_states.shape
    x = hidden_states.reshape(-1, n)
    T = x.shape[0]
    out = pl.pallas_call(
        _moe_kernel,
        grid=(E, NF),
        in_specs=[
            pl.BlockSpec((T, H), lambda e, f: (0, 0)),
            pl.BlockSpec((E, H), lambda e, f: (0, 0)),
            pl.BlockSpec((1, 2, FBLK, H), lambda e, f: (e, 0, f, 0)),
            pl.BlockSpec((1, H, FBLK), lambda e, f: (e, 0, f)),
        ],
        out_specs=pl.BlockSpec((T, H), lambda e, f: (0, 0)),
        out_shape=jax.ShapeDtypeStruct((T, H), jnp.float32),
        scratch_shapes=[pltpu.VMEM((E, T), jnp.float32)],
        compiler_params=pltpu.CompilerParams(
            dimension_semantics=("arbitrary", "arbitrary"),
        ),
    )(x, Wr, W1.reshape(E, 2, FFN, H), W2)
    return out.reshape(s, b, n)
